# baseline (speedup 1.0000x reference)
import functools

import jax
import jax.numpy as jnp
from jax.experimental import pallas as pl
from jax.experimental.pallas import tpu as pltpu

VOCAB = 100000
D = 1024
BATCH = 4
SEQ = 8192
TOT = BATCH * SEQ

K = 256
NBLK = TOT // K


def _tc_body(ids_smem, table_hbm, out_ref, sem):
    i = pl.program_id(0)
    copies = []
    for j in range(K):
        idx = ids_smem[i * K + j]
        cp = pltpu.make_async_copy(table_hbm.at[pl.ds(idx, 1)],
                                   out_ref.at[pl.ds(j, 1)], sem)
        cp.start()
        copies.append(cp)
    for cp in copies:
        cp.wait()


_grid_spec = pltpu.PrefetchScalarGridSpec(
    num_scalar_prefetch=1,
    grid=(NBLK,),
    in_specs=[pl.BlockSpec(memory_space=pltpu.MemorySpace.HBM)],
    out_specs=pl.BlockSpec((K, D), lambda i, ids: (i, 0)),
    scratch_shapes=[pltpu.SemaphoreType.DMA],
)

_tc_gather = pl.pallas_call(
    _tc_body,
    grid_spec=_grid_spec,
    out_shape=jax.ShapeDtypeStruct((TOT, D), jnp.float32),
)


def kernel(input_ids, table):
    ids = input_ids.reshape(TOT).astype(jnp.int32)
    out = _tc_gather(ids, table)
    return out.reshape(BATCH, SEQ, D)


# speedup vs baseline: 1.6383x; 1.6383x over previous
import functools

import jax
import jax.numpy as jnp
from jax import lax
from jax.experimental import pallas as pl
from jax.experimental.pallas import tpu as pltpu
from jax.experimental.pallas import tpu_sc as plsc

VOCAB = 100000
D = 1024
BATCH = 4
SEQ = 8192
TOT = BATCH * SEQ

_info = plsc.get_sparse_core_info()
NC = _info.num_cores
NS = _info.num_subcores
NW = NC * NS
M = 23040
BPW = M // NW
CH = 16
NCHUNK = BPW // CH
NBUF = 4
LOOKAHEAD = 2

NTC = TOT - M
K = 256
NBLK = NTC // K

_mesh = plsc.VectorSubcoreMesh(core_axis_name="c", subcore_axis_name="s")


@functools.partial(
    pl.kernel,
    mesh=_mesh,
    out_type=jax.ShapeDtypeStruct((TOT, D), jnp.float32),
    scratch_types=[
        pltpu.VMEM((NCHUNK, CH), jnp.int32),
        pltpu.VMEM((NBUF, CH, D), jnp.float32),
        pltpu.SemaphoreType.DMA,
        pltpu.SemaphoreType.DMA,
        pltpu.SemaphoreType.DMA,
        pltpu.SemaphoreType.DMA,
        pltpu.SemaphoreType.DMA,
        pltpu.SemaphoreType.DMA,
        pltpu.SemaphoreType.DMA,
        pltpu.SemaphoreType.DMA,
    ],
)
def _sc_embed(idx_hbm, table_hbm, out_hbm, idx_v, bufs,
              g0, g1, g2, g3, s0, s1, s2, s3):
    wid = lax.axis_index("s") * NC + lax.axis_index("c")
    base = wid * BPW
    gsems = (g0, g1, g2, g3)
    ssems = (s0, s1, s2, s3)

    def gather(c, b):
        return pltpu.make_async_copy(table_hbm.at[idx_v.at[c]], bufs.at[b],
                                     gsems[b])

    def store(c, b):
        return pltpu.make_async_copy(
            bufs.at[b], out_hbm.at[pl.ds(base + c * CH, CH)], ssems[b])

    pltpu.sync_copy(idx_hbm.at[wid], idx_v)

    for c in range(LOOKAHEAD):
        gather(c, c % NBUF).start()

    def chunk_body(c, _):
        cn = c + LOOKAHEAD

        @pl.when(cn < NCHUNK)
        def _issue_next():
            @pl.when(cn >= NBUF)
            def _free_buf():
                for b in range(NBUF):

                    @pl.when((cn % NBUF) == b)
                    def _w():
                        store(cn - NBUF, b).wait()

            for b in range(NBUF):

                @pl.when((cn % NBUF) == b)
                def _g():
                    gather(cn, b).start()

        for b in range(NBUF):

            @pl.when((c % NBUF) == b)
            def _cur():
                gather(c, b).wait()
                store(c, b).start()

        return _

    lax.fori_loop(0, NCHUNK, chunk_body, None)

    for c in range(NCHUNK - NBUF, NCHUNK):
        store(c, c % NBUF).wait()


def _tc_body(ids_smem, table_hbm, out_ref, sem):
    i = pl.program_id(0)
    copies = []
    for j in range(K):
        idx = ids_smem[i * K + j]
        cp = pltpu.make_async_copy(table_hbm.at[pl.ds(idx, 1)],
                                   out_ref.at[pl.ds(j, 1)], sem)
        cp.start()
        copies.append(cp)
    for cp in copies:
        cp.wait()


_tc_gather = pl.pallas_call(
    _tc_body,
    grid_spec=pltpu.PrefetchScalarGridSpec(
        num_scalar_prefetch=1,
        grid=(NBLK,),
        in_specs=[pl.BlockSpec(memory_space=pltpu.MemorySpace.HBM)],
        out_specs=pl.BlockSpec((K, D), lambda i, ids: (i, 0)),
        scratch_shapes=[pltpu.SemaphoreType.DMA],
    ),
    out_shape=jax.ShapeDtypeStruct((NTC, D), jnp.float32),
)


def kernel(input_ids, table):
    ids = input_ids.reshape(TOT).astype(jnp.int32)
    ids_sc = ids[:M].reshape(NW, NCHUNK, CH)
    ids_tc = ids[M:]
    sc_full = _sc_embed(ids_sc, table)
    tc_part = _tc_gather(ids_tc, table)
    out = lax.dynamic_update_slice(sc_full, tc_part, (M, 0))
    return out.reshape(BATCH, SEQ, D)


# speedup vs baseline: 2.2908x; 1.3983x over previous
import functools

import jax
import jax.numpy as jnp
from jax import lax
from jax.experimental import pallas as pl
from jax.experimental.pallas import tpu as pltpu
from jax.experimental.pallas import tpu_sc as plsc

VOCAB = 100000
D = 1024
BATCH = 4
SEQ = 8192
TOT = BATCH * SEQ

_info = plsc.get_sparse_core_info()
NC = _info.num_cores
NS = _info.num_subcores
NW = NC * NS
BPW = TOT // NW
WPR = SEQ // BPW
CH = 32
NCHUNK = BPW // CH
NBUF = 3
LOOKAHEAD = 1

_mesh = plsc.VectorSubcoreMesh(core_axis_name="c", subcore_axis_name="s")


@functools.partial(
    pl.kernel,
    mesh=_mesh,
    out_type=jax.ShapeDtypeStruct((TOT, D), jnp.float32),
    scratch_types=[
        pltpu.VMEM((BPW,), jnp.int32),
        pltpu.VMEM((NBUF, CH, D), jnp.float32),
        pltpu.SemaphoreType.DMA,
        pltpu.SemaphoreType.DMA,
        pltpu.SemaphoreType.DMA,
        pltpu.SemaphoreType.DMA,
        pltpu.SemaphoreType.DMA,
        pltpu.SemaphoreType.DMA,
    ],
)
def _sc_embed(idx_hbm, table_hbm, out_hbm, idx_v, bufs,
              g0, g1, g2, s0, s1, s2):
    wid = lax.axis_index("s") * NC + lax.axis_index("c")
    base = wid * BPW
    gsems = (g0, g1, g2)
    ssems = (s0, s1, s2)

    def gather(c, b):
        return pltpu.make_async_copy(
            table_hbm.at[idx_v.at[pl.ds(c * CH, CH)]], bufs.at[b], gsems[b])

    def store(c, b):
        return pltpu.make_async_copy(
            bufs.at[b], out_hbm.at[pl.ds(base + c * CH, CH)], ssems[b])

    pltpu.sync_copy(idx_hbm.at[wid // WPR, pl.ds((wid % WPR) * BPW, BPW)],
                    idx_v)

    for c in range(LOOKAHEAD):
        gather(c, c % NBUF).start()

    def chunk_body(c, _):
        cn = c + LOOKAHEAD

        @pl.when(cn < NCHUNK)
        def _issue_next():
            for b in range(NBUF):

                @pl.when((cn % NBUF) == b)
                def _g():
                    @pl.when(cn >= NBUF)
                    def _free_buf():
                        store(cn - NBUF, b).wait()

                    gather(cn, b).start()

        for b in range(NBUF):

            @pl.when((c % NBUF) == b)
            def _cur():
                gather(c, b).wait()
                store(c, b).start()

        return _

    lax.fori_loop(0, NCHUNK, chunk_body, None)

    for c in range(NCHUNK - NBUF, NCHUNK):
        store(c, c % NBUF).wait()


def kernel(input_ids, table):
    out = _sc_embed(input_ids.astype(jnp.int32), table)
    return out.reshape(BATCH, SEQ, D)
